# baseline (speedup 1.0000x reference)
import functools

import jax
import jax.numpy as jnp
from jax import lax
from jax.experimental import pallas as pl
from jax.experimental.pallas import tpu as pltpu
from jax.experimental.pallas import tpu_sc as plsc

MAX_SEQ = 8192
D_MODEL = 768
BATCH = 4

NUM_CORES = 2
NUM_SUBCORES = 16
NUM_WORKERS = NUM_CORES * NUM_SUBCORES
S_PER_W = MAX_SEQ // NUM_WORKERS
CHUNK = 64
N_CHUNKS = S_PER_W // CHUNK

_MESH = plsc.VectorSubcoreMesh(core_axis_name="c", subcore_axis_name="s")


@functools.partial(
    pl.kernel,
    mesh=_MESH,
    out_type=jax.ShapeDtypeStruct((BATCH * MAX_SEQ, D_MODEL), jnp.float32),
    scratch_types=[
        pltpu.VMEM((S_PER_W,), jnp.int32),
        pltpu.VMEM((CHUNK, D_MODEL), jnp.float32),
        pltpu.VMEM((CHUNK, D_MODEL), jnp.float32),
        pltpu.SemaphoreType.DMA,
        pltpu.SemaphoreType.DMA,
        pltpu.SemaphoreType.DMA,
    ],
)
def _pe_lookup_tile(emb_hbm, pe_hbm, out_hbm, idx_v, rows0, rows1,
                    gsem, wsem0, wsem1):
    wid = lax.axis_index("s") * NUM_CORES + lax.axis_index("c")
    base = wid * S_PER_W
    rows = (rows0, rows1)
    wsem = (wsem0, wsem1)
    pltpu.sync_copy(pe_hbm.at[pl.ds(base, S_PER_W)], idx_v)
    pending = {0: [], 1: []}
    for i in range(N_CHUNKS):
        cur = i & 1
        off = base + i * CHUNK
        for c in pending[cur]:
            c.wait()
        pending[cur] = []
        g = pltpu.async_copy(
            emb_hbm.at[idx_v.at[pl.ds(i * CHUNK, CHUNK)]], rows[cur], gsem)
        g.wait()
        for b in range(BATCH):
            pending[cur].append(pltpu.async_copy(
                rows[cur], out_hbm.at[pl.ds(b * MAX_SEQ + off, CHUNK)],
                wsem[cur]))
    for cur in (0, 1):
        for c in pending[cur]:
            c.wait()


def kernel(x, emb, pe):
    del x
    pe_flat = pe.reshape(MAX_SEQ).astype(jnp.int32)
    out = _pe_lookup_tile(emb, pe_flat)
    return out.reshape(BATCH, MAX_SEQ, D_MODEL)


# speedup vs baseline: 1.0170x; 1.0170x over previous
import functools

import jax
import jax.numpy as jnp
from jax import lax
from jax.experimental import pallas as pl
from jax.experimental.pallas import tpu as pltpu
from jax.experimental.pallas import tpu_sc as plsc

MAX_SEQ = 8192
D_MODEL = 768
BATCH = 4

NUM_CORES = 2
NUM_SUBCORES = 16
NUM_WORKERS = NUM_CORES * NUM_SUBCORES
S_PER_W = MAX_SEQ // NUM_WORKERS
CHUNK = 128
N_CHUNKS = S_PER_W // CHUNK

_MESH = plsc.VectorSubcoreMesh(core_axis_name="c", subcore_axis_name="s")


@functools.partial(
    pl.kernel,
    mesh=_MESH,
    out_type=jax.ShapeDtypeStruct((BATCH * MAX_SEQ, D_MODEL), jnp.float32),
    scratch_types=[
        pltpu.VMEM((S_PER_W,), jnp.int32),
        pltpu.VMEM((CHUNK, D_MODEL), jnp.float32),
        pltpu.SemaphoreType.DMA,
        pltpu.SemaphoreType.DMA,
    ],
)
def _pe_lookup_tile(emb_hbm, pe_hbm, out_hbm, idx_v, rows_v, gsem, wsem):
    wid = lax.axis_index("s") * NUM_CORES + lax.axis_index("c")
    base = wid * S_PER_W
    pltpu.sync_copy(pe_hbm.at[pl.ds(base, S_PER_W)], idx_v)
    pending = []
    for i in range(N_CHUNKS):
        off = base + i * CHUNK
        for c in pending:
            c.wait()
        pending = []
        pltpu.async_copy(
            emb_hbm.at[idx_v.at[pl.ds(i * CHUNK, CHUNK)]], rows_v, gsem
        ).wait()
        for b in range(BATCH):
            pending.append(pltpu.async_copy(
                rows_v, out_hbm.at[pl.ds(b * MAX_SEQ + off, CHUNK)], wsem))
    for c in pending:
        c.wait()


def kernel(x, emb, pe):
    del x
    pe_flat = pe.reshape(MAX_SEQ).astype(jnp.int32)
    out = _pe_lookup_tile(emb, pe_flat)
    return out.reshape(BATCH, MAX_SEQ, D_MODEL)
